# baseline (speedup 1.0000x reference)
import functools

import jax
import jax.numpy as jnp
import numpy as np
from jax.experimental import pallas as pl
from jax.experimental.pallas import tpu as pltpu

_C = 19
_SUB = 256
_LANE = 1024
_IGNORE = 255


def _ohem_kernel(logits_ref, labels_ref, out_ref, loss_scr, acc_e, acc_l,
                 cnt_acc, sum_acc, vld_acc, *, n_b, n_min_static, thresh):
    b = pl.program_id(0)
    c = pl.program_id(1)

    s = logits_ref[0, 0]
    lab = labels_ref[0]

    e = jnp.exp(s)
    sel = lab == c

    @pl.when(c == 0)
    def _first_class():
        acc_e[...] = e
        acc_l[...] = jnp.where(sel, s, 0.0)

    @pl.when(c != 0)
    def _accum_class():
        acc_e[...] += e
        acc_l[...] = jnp.where(sel, s, acc_l[...])

    @pl.when(jnp.logical_and(b == 0, c == 0))
    def _init():
        cnt_acc[...] = jnp.zeros_like(cnt_acc)
        sum_acc[...] = jnp.zeros_like(sum_acc)
        vld_acc[...] = jnp.zeros_like(vld_acc)

    @pl.when(c == _C - 1)
    def _finish_batch():
        valid = lab != _IGNORE
        loss = jnp.where(valid, jnp.log(acc_e[...]) - acc_l[...], 0.0)
        loss_scr[pl.ds(b, 1)] = loss[None]
        mask = loss > thresh
        cnt_acc[...] += mask.astype(jnp.int32)
        sum_acc[...] += jnp.where(mask, loss, 0.0)
        vld_acc[...] += valid.astype(jnp.int32)

        @pl.when(b == n_b - 1)
        def _finalize():
            n_hard = jnp.sum(cnt_acc[...])
            hard_sum = jnp.sum(sum_acc[...])
            n_min = jnp.sum(vld_acc[...]) // 16
            few = n_hard < n_min

            @pl.when(jnp.logical_not(few))
            def _many():
                out_ref[0] = hard_sum / n_hard.astype(jnp.float32)

            @pl.when(few)
            def _few():
                k = n_min_static
                lv = loss_scr[...]
                bits = jax.lax.bitcast_convert_type(lv, jnp.int32)

                def body(j, ans):
                    trial = ans | (1 << (30 - j))
                    cc = jnp.sum((bits > trial).astype(jnp.int32))
                    return jnp.where(cc >= k, trial, ans)

                ans = jax.lax.fori_loop(0, 31, body, jnp.int32(0))
                c0 = jnp.sum((bits > 0).astype(jnp.int32))
                tbits = jnp.where(c0 >= k, ans + 1, 0)
                t = jax.lax.bitcast_convert_type(tbits, jnp.float32)
                gt = bits > tbits
                cnt_gt = jnp.sum(gt.astype(jnp.int32))
                sum_gt = jnp.sum(jnp.where(gt, lv, 0.0))
                out_ref[0] = (
                    sum_gt + (k - cnt_gt).astype(jnp.float32) * t
                ) / jnp.float32(k)


def kernel(logits, labels):
    b, c, h, w = logits.shape
    npix = b * h * w
    thresh = float(-np.log(np.float32(0.7)))

    logits4 = logits.reshape(b, c, _SUB, _LANE)
    labels3 = labels.reshape(b, _SUB, _LANE)

    body = functools.partial(
        _ohem_kernel,
        n_b=b,
        n_min_static=npix // 16,
        thresh=thresh,
    )

    out = pl.pallas_call(
        body,
        grid=(b, c),
        in_specs=[
            pl.BlockSpec((1, 1, _SUB, _LANE), lambda i, j: (i, j, 0, 0)),
            pl.BlockSpec((1, _SUB, _LANE), lambda i, j: (i, 0, 0)),
        ],
        out_specs=pl.BlockSpec(memory_space=pltpu.SMEM),
        out_shape=jax.ShapeDtypeStruct((1,), jnp.float32),
        scratch_shapes=[
            pltpu.VMEM((b, _SUB, _LANE), jnp.float32),
            pltpu.VMEM((_SUB, _LANE), jnp.float32),
            pltpu.VMEM((_SUB, _LANE), jnp.float32),
            pltpu.VMEM((_SUB, _LANE), jnp.int32),
            pltpu.VMEM((_SUB, _LANE), jnp.float32),
            pltpu.VMEM((_SUB, _LANE), jnp.int32),
        ],
    )(logits4, labels3)
    return out[0]


# speedup vs baseline: 1.3560x; 1.3560x over previous
import functools

import jax
import jax.numpy as jnp
import numpy as np
from jax.experimental import pallas as pl
from jax.experimental.pallas import tpu as pltpu

_C = 19
_SUB = 128
_LANE = 1024
_CHUNK = _SUB * _LANE
_IGNORE = 255


def _ohem_kernel(logits_ref, labels_ref, out_ref, loss_scr, cnt_acc, sum_acc,
                 vld_acc, *, n_steps, n_min_static, thresh):
    i = pl.program_id(0)

    lab = labels_ref[0]

    acc_e = jnp.zeros((_SUB, _LANE), jnp.float32)
    acc_l = jnp.zeros((_SUB, _LANE), jnp.float32)
    for c in range(_C):
        s = logits_ref[0, c, 0]
        acc_e += jnp.exp(s)
        acc_l = jnp.where(lab == c, s, acc_l)

    valid = lab != _IGNORE
    loss = jnp.where(valid, jnp.log(acc_e) - acc_l, 0.0)

    loss_scr[pl.ds(i, 1)] = loss[None]

    mask = loss > thresh

    @pl.when(i == 0)
    def _init():
        cnt_acc[...] = jnp.zeros_like(cnt_acc)
        sum_acc[...] = jnp.zeros_like(sum_acc)
        vld_acc[...] = jnp.zeros_like(vld_acc)

    cnt_acc[...] += mask.astype(jnp.int32)
    sum_acc[...] += jnp.where(mask, loss, 0.0)
    vld_acc[...] += valid.astype(jnp.int32)

    @pl.when(i == n_steps - 1)
    def _finalize():
        n_hard = jnp.sum(cnt_acc[...])
        hard_sum = jnp.sum(sum_acc[...])
        n_min = jnp.sum(vld_acc[...]) // 16
        few = n_hard < n_min

        @pl.when(jnp.logical_not(few))
        def _many():
            out_ref[0] = hard_sum / n_hard.astype(jnp.float32)

        @pl.when(few)
        def _few():
            k = n_min_static
            lv = loss_scr[...]
            bits = jax.lax.bitcast_convert_type(lv, jnp.int32)

            def body(j, ans):
                trial = ans | (1 << (30 - j))
                c = jnp.sum((bits > trial).astype(jnp.int32))
                return jnp.where(c >= k, trial, ans)

            ans = jax.lax.fori_loop(0, 31, body, jnp.int32(0))
            c0 = jnp.sum((bits > 0).astype(jnp.int32))
            tbits = jnp.where(c0 >= k, ans + 1, 0)
            t = jax.lax.bitcast_convert_type(tbits, jnp.float32)
            gt = bits > tbits
            cnt_gt = jnp.sum(gt.astype(jnp.int32))
            sum_gt = jnp.sum(jnp.where(gt, lv, 0.0))
            out_ref[0] = (
                sum_gt + (k - cnt_gt).astype(jnp.float32) * t
            ) / jnp.float32(k)


def kernel(logits, labels):
    b, c, h, w = logits.shape
    npix = b * h * w
    n_steps = npix // _CHUNK
    chunks_per_b = (h * w) // _CHUNK
    thresh = float(-np.log(np.float32(0.7)))

    logits5 = logits.reshape(b, c, chunks_per_b, _SUB, _LANE)
    labels3 = labels.reshape(n_steps, _SUB, _LANE)

    body = functools.partial(
        _ohem_kernel,
        n_steps=n_steps,
        n_min_static=npix // 16,
        thresh=thresh,
    )

    out = pl.pallas_call(
        body,
        grid=(n_steps,),
        in_specs=[
            pl.BlockSpec((1, c, 1, _SUB, _LANE),
                         lambda i: (i // chunks_per_b, 0, i % chunks_per_b,
                                    0, 0)),
            pl.BlockSpec((1, _SUB, _LANE), lambda i: (i, 0, 0)),
        ],
        out_specs=pl.BlockSpec(memory_space=pltpu.SMEM),
        out_shape=jax.ShapeDtypeStruct((1,), jnp.float32),
        scratch_shapes=[
            pltpu.VMEM((n_steps, _SUB, _LANE), jnp.float32),
            pltpu.VMEM((_SUB, _LANE), jnp.int32),
            pltpu.VMEM((_SUB, _LANE), jnp.float32),
            pltpu.VMEM((_SUB, _LANE), jnp.int32),
        ],
    )(logits5, labels3)
    return out[0]


# speedup vs baseline: 1.3574x; 1.0011x over previous
import functools

import jax
import jax.numpy as jnp
import numpy as np
from jax.experimental import pallas as pl
from jax.experimental.pallas import tpu as pltpu

_C = 19
_SUB = 128
_HALF = _SUB // 2
_LANE = 1024
_CHUNK = _SUB * _LANE
_IGNORE = 255


def _ohem_kernel(logits_a, logits_b, labels_ref, out_ref, loss_scr, cnt_acc,
                 sum_acc, vld_acc, *, n_steps, n_min_static, thresh):
    i = pl.program_id(0)

    lab = labels_ref[0]

    @pl.when(i == 0)
    def _init():
        cnt_acc[...] = jnp.zeros_like(cnt_acc)
        sum_acc[...] = jnp.zeros_like(sum_acc)
        vld_acc[...] = jnp.zeros_like(vld_acc)

    for h, ref in ((0, logits_a), (1, logits_b)):
        labh = lab[h * _HALF:(h + 1) * _HALF]
        acc_e = jnp.zeros((_HALF, _LANE), jnp.float32)
        acc_l = jnp.zeros((_HALF, _LANE), jnp.float32)
        for c in range(_C):
            s = ref[0, c, 0, 0]
            acc_e += jnp.exp(s)
            acc_l = jnp.where(labh == c, s, acc_l)

        valid = labh != _IGNORE
        loss = jnp.where(valid, jnp.log(acc_e) - acc_l, 0.0)
        loss_scr[pl.ds(i, 1), h * _HALF:(h + 1) * _HALF] = loss[None]
        mask = loss > thresh
        cnt_acc[...] += mask.astype(jnp.int32)
        sum_acc[...] += jnp.where(mask, loss, 0.0)
        vld_acc[...] += valid.astype(jnp.int32)

    @pl.when(i == n_steps - 1)
    def _finalize():
        n_hard = jnp.sum(cnt_acc[...])
        hard_sum = jnp.sum(sum_acc[...])
        n_min = jnp.sum(vld_acc[...]) // 16
        few = n_hard < n_min

        @pl.when(jnp.logical_not(few))
        def _many():
            out_ref[0] = hard_sum / n_hard.astype(jnp.float32)

        @pl.when(few)
        def _few():
            k = n_min_static
            lv = loss_scr[...]
            bits = jax.lax.bitcast_convert_type(lv, jnp.int32)

            def body(j, ans):
                trial = ans | (1 << (30 - j))
                c = jnp.sum((bits > trial).astype(jnp.int32))
                return jnp.where(c >= k, trial, ans)

            ans = jax.lax.fori_loop(0, 31, body, jnp.int32(0))
            c0 = jnp.sum((bits > 0).astype(jnp.int32))
            tbits = jnp.where(c0 >= k, ans + 1, 0)
            t = jax.lax.bitcast_convert_type(tbits, jnp.float32)
            gt = bits > tbits
            cnt_gt = jnp.sum(gt.astype(jnp.int32))
            sum_gt = jnp.sum(jnp.where(gt, lv, 0.0))
            out_ref[0] = (
                sum_gt + (k - cnt_gt).astype(jnp.float32) * t
            ) / jnp.float32(k)


def kernel(logits, labels):
    b, c, h, w = logits.shape
    npix = b * h * w
    n_steps = npix // _CHUNK
    chunks_per_b = (h * w) // _CHUNK
    thresh = float(-np.log(np.float32(0.7)))

    logits6 = logits.reshape(b, c, chunks_per_b, 2, _HALF, _LANE)
    labels3 = labels.reshape(n_steps, _SUB, _LANE)

    body = functools.partial(
        _ohem_kernel,
        n_steps=n_steps,
        n_min_static=npix // 16,
        thresh=thresh,
    )

    spec_half = lambda half: pl.BlockSpec(
        (1, c, 1, 1, _HALF, _LANE),
        lambda i: (i // chunks_per_b, 0, i % chunks_per_b, half, 0, 0))

    out = pl.pallas_call(
        body,
        grid=(n_steps,),
        in_specs=[
            spec_half(0),
            spec_half(1),
            pl.BlockSpec((1, _SUB, _LANE), lambda i: (i, 0, 0)),
        ],
        out_specs=pl.BlockSpec(memory_space=pltpu.SMEM),
        out_shape=jax.ShapeDtypeStruct((1,), jnp.float32),
        scratch_shapes=[
            pltpu.VMEM((n_steps, _SUB, _LANE), jnp.float32),
            pltpu.VMEM((_HALF, _LANE), jnp.int32),
            pltpu.VMEM((_HALF, _LANE), jnp.float32),
            pltpu.VMEM((_HALF, _LANE), jnp.int32),
        ],
    )(logits6, logits6, labels3)
    return out[0]
